# baseline (speedup 1.0000x reference)
import functools

import jax
import jax.numpy as jnp
import numpy as np
from jax import lax
from jax.experimental import pallas as pl
from jax.experimental.pallas import tpu as pltpu
from jax.experimental.pallas import tpu_sc as plsc

SEQ = 2048
BATCH = 64
DIM = 128
LANES = 16
VPR = DIM // LANES

NC = 2
NS = 16
NW = NC * NS

N = SEQ * BATCH
ROWS_W = N // NW
POS_W = SEQ // NW
CHUNK = 64
POS_CHUNK = CHUNK // BATCH
NCHUNK = ROWS_W // CHUNK
NBUF = 8
UNROLL = 4


def _make_pe():
    pos = np.arange(SEQ, dtype=np.float32)[:, None]
    div_term = 1.0 / np.power(
        10000.0, np.arange(0, DIM * 2, 2, dtype=np.float32) / DIM
    )
    pe = pos * div_term[None, :]
    pe[:, 0::2] = np.sin(pe[:, 0::2])
    pe[:, 1::2] = np.cos(pe[:, 1::2])
    return jnp.asarray(pe)


@functools.partial(
    pl.kernel,
    mesh=plsc.VectorSubcoreMesh(core_axis_name="c", subcore_axis_name="s"),
    out_type=jax.ShapeDtypeStruct((N, DIM), jnp.float32),
    scratch_types=(
        [pltpu.VMEM((ROWS_W,), jnp.int32)]
        + [pltpu.VMEM((ROWS_W // DIM, DIM), jnp.int32)]
        + [pltpu.VMEM((CHUNK, DIM), jnp.float32) for _ in range(NBUF)]
        + [pltpu.VMEM((POS_W, DIM), jnp.float32)]
        + [pltpu.SemaphoreType.DMA for _ in range(2 * NBUF)]
    ),
)
def _emb_lookup(idx_hbm, table_hbm, pe_hbm, out_hbm, idx_v, idx2, *rest):
    rows = rest[:NBUF]
    pe_v = rest[NBUF]
    gsem = rest[NBUF + 1:NBUF + 1 + NBUF]
    wsem = rest[NBUF + 1 + NBUF:]

    wid = lax.axis_index("s") * NC + lax.axis_index("c")
    base = wid * ROWS_W
    IDX_ROWS = ROWS_W // DIM
    pltpu.sync_copy(idx_hbm.at[pl.ds(wid * IDX_ROWS, IDX_ROWS)], idx2)
    pltpu.sync_copy(pe_hbm.at[pl.ds(wid * POS_W, POS_W)], pe_v)

    def repack_body(r, _):
        for k in range(VPR):
            idx_v[pl.ds(r * DIM + k * LANES, LANES)] = idx2[
                r, pl.ds(k * LANES, LANES)
            ]
        return 0

    lax.fori_loop(0, IDX_ROWS, repack_body, 0)

    def start_gather(c, b):
        return pltpu.async_copy(
            table_hbm.at[idx_v.at[pl.ds(c * CHUNK, CHUNK)]], rows[b], gsem[b]
        )

    def start_wb(c, b):
        return pltpu.async_copy(
            rows[b], out_hbm.at[pl.ds(base + c * CHUNK, CHUNK)], wsem[b]
        )

    def add_pe(c, b):
        def pos_body(p, _):
            lp = c * POS_CHUNK + p
            pe_vecs = tuple(
                pe_v[lp, pl.ds(v * LANES, LANES)] for v in range(VPR)
            )

            def row_body(r, _):
                row = p * BATCH + r * UNROLL
                for u in range(UNROLL):
                    for v in range(VPR):
                        plsc.addupdate(
                            rows[b].at[row + u, pl.ds(v * LANES, LANES)],
                            pe_vecs[v],
                        )
                return 0

            lax.fori_loop(0, BATCH // UNROLL, row_body, 0)
            return 0

        lax.fori_loop(0, POS_CHUNK, pos_body, 0)

    for b in range(NBUF - 1):
        start_gather(b, b)

    def ring_body(t, _):
        for b in range(NBUF):
            c = t * NBUF + b
            pltpu.make_async_copy(
                table_hbm.at[idx_v.at[pl.ds(0, CHUNK)]], rows[b], gsem[b]
            ).wait()
            add_pe(c, b)
            start_wb(c, b)
            nb = (b + NBUF - 1) % NBUF
            nc = c + NBUF - 1

            @pl.when(nc < NCHUNK)
            def _():
                @pl.when(nc >= NBUF)
                def _():
                    pltpu.make_async_copy(
                        rows[nb], out_hbm.at[pl.ds(base, CHUNK)], wsem[nb]
                    ).wait()

                start_gather(nc, nb)

        return 0

    lax.fori_loop(0, NCHUNK // NBUF, ring_body, 0)

    for b in range(NBUF):
        pltpu.make_async_copy(
            rows[b], out_hbm.at[pl.ds(base, CHUNK)], wsem[b]
        ).wait()


def kernel(input, table):
    idx = input.reshape(N // DIM, DIM)
    pe = _make_pe()
    out = _emb_lookup(idx, table, pe)
    return out.reshape(SEQ, BATCH, DIM)


# speedup vs baseline: 1.0007x; 1.0007x over previous
import functools

import jax
import jax.numpy as jnp
import numpy as np
from jax import lax
from jax.experimental import pallas as pl
from jax.experimental.pallas import tpu as pltpu
from jax.experimental.pallas import tpu_sc as plsc

SEQ = 2048
BATCH = 64
DIM = 128
LANES = 16
VPR = DIM // LANES

NC = 2
NS = 16
NW = NC * NS

N = SEQ * BATCH
ROWS_W = N // NW
POS_W = SEQ // NW
CHUNK = 64
POS_CHUNK = CHUNK // BATCH
NCHUNK = ROWS_W // CHUNK
NBUF = 8
UNROLL = 4


def _make_pe():
    pos = np.arange(SEQ, dtype=np.float32)[:, None]
    div_term = 1.0 / np.power(
        10000.0, np.arange(0, DIM * 2, 2, dtype=np.float32) / DIM
    )
    pe = pos * div_term[None, :]
    pe[:, 0::2] = np.sin(pe[:, 0::2])
    pe[:, 1::2] = np.cos(pe[:, 1::2])
    return jnp.asarray(pe)


@functools.partial(
    pl.kernel,
    mesh=plsc.VectorSubcoreMesh(core_axis_name="c", subcore_axis_name="s"),
    out_type=jax.ShapeDtypeStruct((N, DIM), jnp.float32),
    scratch_types=(
        [pltpu.VMEM((ROWS_W,), jnp.int32)]
        + [pltpu.VMEM((CHUNK, DIM), jnp.float32) for _ in range(NBUF)]
        + [pltpu.VMEM((POS_W, DIM), jnp.float32)]
        + [pltpu.SemaphoreType.DMA for _ in range(2 * NBUF)]
    ),
)
def _emb_lookup(idx_hbm, table_hbm, pe_hbm, out_hbm, idx_v, *rest):
    rows = rest[:NBUF]
    pe_v = rest[NBUF]
    gsem = rest[NBUF + 1:NBUF + 1 + NBUF]
    wsem = rest[NBUF + 1 + NBUF:]

    wid = lax.axis_index("s") * NC + lax.axis_index("c")
    base = wid * ROWS_W
    pltpu.sync_copy(idx_hbm.at[pl.ds(base, ROWS_W)], idx_v)

    def start_gather(c, b):
        return pltpu.async_copy(
            table_hbm.at[idx_v.at[pl.ds(c * CHUNK, CHUNK)]], rows[b], gsem[b]
        )

    def start_wb(c, b):
        return pltpu.async_copy(
            rows[b], out_hbm.at[pl.ds(base + c * CHUNK, CHUNK)], wsem[b]
        )

    def add_pe(c, b):
        def pos_body(p, _):
            lp = c * POS_CHUNK + p
            pe_vecs = tuple(
                pe_v[lp, pl.ds(v * LANES, LANES)] for v in range(VPR)
            )

            def row_body(r, _):
                row = p * BATCH + r * UNROLL
                for u in range(UNROLL):
                    for v in range(VPR):
                        plsc.addupdate(
                            rows[b].at[row + u, pl.ds(v * LANES, LANES)],
                            pe_vecs[v],
                        )
                return 0

            lax.fori_loop(0, BATCH // UNROLL, row_body, 0)
            return 0

        lax.fori_loop(0, POS_CHUNK, pos_body, 0)

    for b in range(NBUF - 1):
        start_gather(b, b)
    pltpu.sync_copy(pe_hbm.at[pl.ds(wid * POS_W, POS_W)], pe_v)

    def ring_body(t, _):
        for b in range(NBUF):
            c = t * NBUF + b
            pltpu.make_async_copy(
                table_hbm.at[idx_v.at[pl.ds(0, CHUNK)]], rows[b], gsem[b]
            ).wait()
            add_pe(c, b)
            start_wb(c, b)
            nb = (b + NBUF - 1) % NBUF
            nc = c + NBUF - 1

            @pl.when(nc < NCHUNK)
            def _():
                @pl.when(nc >= NBUF)
                def _():
                    pltpu.make_async_copy(
                        rows[nb], out_hbm.at[pl.ds(base, CHUNK)], wsem[nb]
                    ).wait()

                start_gather(nc, nb)

        return 0

    lax.fori_loop(0, NCHUNK // NBUF, ring_body, 0)

    for b in range(NBUF):
        pltpu.make_async_copy(
            rows[b], out_hbm.at[pl.ds(base, CHUNK)], wsem[b]
        ).wait()


def kernel(input, table):
    idx = input.reshape(N)
    pe = _make_pe()
    out = _emb_lookup(idx, table, pe)
    return out.reshape(SEQ, BATCH, DIM)


# speedup vs baseline: 1.0008x; 1.0001x over previous
import functools

import jax
import jax.numpy as jnp
import numpy as np
from jax import lax
from jax.experimental import pallas as pl
from jax.experimental.pallas import tpu as pltpu
from jax.experimental.pallas import tpu_sc as plsc

SEQ = 2048
BATCH = 64
DIM = 128
LANES = 16
VPR = DIM // LANES

NC = 2
NS = 16
NW = NC * NS

N = SEQ * BATCH
ROWS_W = N // NW
POS_W = SEQ // NW
CHUNK = 64
POS_CHUNK = CHUNK // BATCH
NCHUNK = ROWS_W // CHUNK
NBUF = 8
UNROLL = 4


def _make_pe():
    pos = np.arange(SEQ, dtype=np.float32)[:, None]
    div_term = 1.0 / np.power(
        10000.0, np.arange(0, DIM * 2, 2, dtype=np.float32) / DIM
    )
    pe = pos * div_term[None, :]
    pe[:, 0::2] = np.sin(pe[:, 0::2])
    pe[:, 1::2] = np.cos(pe[:, 1::2])
    return jnp.asarray(pe)


@functools.partial(
    pl.kernel,
    mesh=plsc.VectorSubcoreMesh(core_axis_name="c", subcore_axis_name="s"),
    out_type=jax.ShapeDtypeStruct((N, DIM), jnp.float32),
    compiler_params=pltpu.CompilerParams(use_tc_tiling_on_sc=True),
    scratch_types=(
        [pltpu.VMEM((ROWS_W,), jnp.int32)]
        + [pltpu.VMEM((CHUNK, DIM), jnp.float32) for _ in range(NBUF)]
        + [pltpu.VMEM((POS_W, DIM), jnp.float32)]
        + [pltpu.SemaphoreType.DMA for _ in range(2 * NBUF)]
    ),
)
def _emb_lookup(idx_hbm, table_hbm, pe_hbm, out_hbm, idx_v, *rest):
    rows = rest[:NBUF]
    pe_v = rest[NBUF]
    gsem = rest[NBUF + 1:NBUF + 1 + NBUF]
    wsem = rest[NBUF + 1 + NBUF:]

    wid = lax.axis_index("s") * NC + lax.axis_index("c")
    base = wid * ROWS_W
    pltpu.sync_copy(idx_hbm.at[pl.ds(base, ROWS_W)], idx_v)

    def start_gather(c, b):
        return pltpu.async_copy(
            table_hbm.at[idx_v.at[pl.ds(c * CHUNK, CHUNK)]], rows[b], gsem[b]
        )

    def start_wb(c, b):
        return pltpu.async_copy(
            rows[b], out_hbm.at[pl.ds(base + c * CHUNK, CHUNK)], wsem[b]
        )

    def add_pe(c, b):
        def pos_body(p, _):
            lp = c * POS_CHUNK + p
            pe_vecs = tuple(
                pe_v[lp, pl.ds(v * LANES, LANES)] for v in range(VPR)
            )

            def row_body(r, _):
                row = p * BATCH + r * UNROLL
                for u in range(UNROLL):
                    for v in range(VPR):
                        plsc.addupdate(
                            rows[b].at[row + u, pl.ds(v * LANES, LANES)],
                            pe_vecs[v],
                        )
                return 0

            lax.fori_loop(0, BATCH // UNROLL, row_body, 0)
            return 0

        lax.fori_loop(0, POS_CHUNK, pos_body, 0)

    for b in range(NBUF - 1):
        start_gather(b, b)
    pltpu.sync_copy(pe_hbm.at[pl.ds(wid * POS_W, POS_W)], pe_v)

    def ring_body(t, _):
        for b in range(NBUF):
            c = t * NBUF + b
            pltpu.make_async_copy(
                table_hbm.at[idx_v.at[pl.ds(0, CHUNK)]], rows[b], gsem[b]
            ).wait()
            add_pe(c, b)
            start_wb(c, b)
            nb = (b + NBUF - 1) % NBUF
            nc = c + NBUF - 1

            @pl.when(nc < NCHUNK)
            def _():
                @pl.when(nc >= NBUF)
                def _():
                    pltpu.make_async_copy(
                        rows[nb], out_hbm.at[pl.ds(base, CHUNK)], wsem[nb]
                    ).wait()

                start_gather(nc, nb)

        return 0

    lax.fori_loop(0, NCHUNK // NBUF, ring_body, 0)

    for b in range(NBUF):
        pltpu.make_async_copy(
            rows[b], out_hbm.at[pl.ds(base, CHUNK)], wsem[b]
        ).wait()


def kernel(input, table):
    idx = input.reshape(N)
    pe = _make_pe()
    out = _emb_lookup(idx, table, pe)
    return out.reshape(SEQ, BATCH, DIM)


# speedup vs baseline: 1.0011x; 1.0003x over previous
import functools

import jax
import jax.numpy as jnp
import numpy as np
from jax import lax
from jax.experimental import pallas as pl
from jax.experimental.pallas import tpu as pltpu
from jax.experimental.pallas import tpu_sc as plsc

SEQ = 2048
BATCH = 64
DIM = 128
LANES = 16
VPR = DIM // LANES

NC = 2
NS = 16
NW = NC * NS

N = SEQ * BATCH
ROWS_W = N // NW
POS_W = SEQ // NW
CHUNK = 64
POS_CHUNK = CHUNK // BATCH
NCHUNK = ROWS_W // CHUNK
NBUF = 8
UNROLL = 4


def _make_pe():
    pos = np.arange(SEQ, dtype=np.float32)[:, None]
    div_term = 1.0 / np.power(
        10000.0, np.arange(0, DIM * 2, 2, dtype=np.float32) / DIM
    )
    pe = pos * div_term[None, :]
    pe[:, 0::2] = np.sin(pe[:, 0::2])
    pe[:, 1::2] = np.cos(pe[:, 1::2])
    return jnp.asarray(pe)


@functools.partial(
    pl.kernel,
    mesh=plsc.VectorSubcoreMesh(core_axis_name="c", subcore_axis_name="s"),
    out_type=jax.ShapeDtypeStruct((N, DIM), jnp.float32),
    scratch_types=(
        [pltpu.VMEM((ROWS_W,), jnp.int32)]
        + [pltpu.VMEM((CHUNK, DIM), jnp.float32) for _ in range(NBUF)]
        + [pltpu.VMEM((POS_W, DIM), jnp.float32)]
        + [pltpu.SemaphoreType.DMA for _ in range(2 * NBUF)]
    ),
)
def _emb_lookup(idx_hbm, table_hbm, pe_hbm, out_hbm, idx_v, *rest):
    rows = rest[:NBUF]
    pe_v = rest[NBUF]
    gsem = rest[NBUF + 1:NBUF + 1 + NBUF]
    wsem = rest[NBUF + 1 + NBUF:]

    wid = lax.axis_index("s") * NC + lax.axis_index("c")
    base = wid * ROWS_W
    pltpu.sync_copy(idx_hbm.at[pl.ds(base, ROWS_W)], idx_v)

    def start_gather(c, b):
        return pltpu.async_copy(
            table_hbm.at[idx_v.at[pl.ds(c * CHUNK, CHUNK)]], rows[b], gsem[b]
        )

    def start_wb(c, b):
        return pltpu.async_copy(
            rows[b], out_hbm.at[pl.ds(base + c * CHUNK, CHUNK)], wsem[b]
        )

    def add_pe(c, b):
        def pos_body(p, _):
            lp = c * POS_CHUNK + p
            pe_vecs = tuple(
                pe_v[lp, pl.ds(v * LANES, LANES)] for v in range(VPR)
            )

            def row_body(r, _):
                row = p * BATCH + r * UNROLL
                for u in range(UNROLL):
                    for v in range(VPR):
                        plsc.addupdate(
                            rows[b].at[row + u, pl.ds(v * LANES, LANES)],
                            pe_vecs[v],
                        )
                return 0

            lax.fori_loop(0, BATCH // UNROLL, row_body, 0)
            return 0

        lax.fori_loop(0, POS_CHUNK, pos_body, 0)

    for b in range(NBUF - 1):
        start_gather(b, b)
    pltpu.sync_copy(pe_hbm.at[pl.ds(wid * POS_W, POS_W)], pe_v)

    def ring_body(t, _):
        for b in range(NBUF):
            c = t * NBUF + b
            pltpu.make_async_copy(
                table_hbm.at[idx_v.at[pl.ds(0, CHUNK)]], rows[b], gsem[b]
            ).wait()
            add_pe(c, b)
            start_wb(c, b)
            nb = (b + NBUF - 1) % NBUF
            nc = c + NBUF - 1

            @pl.when(nc < NCHUNK)
            def _():
                @pl.when(nc >= NBUF)
                def _():
                    pltpu.make_async_copy(
                        rows[nb], out_hbm.at[pl.ds(base, CHUNK)], wsem[nb]
                    ).wait()

                start_gather(nc, nb)

        return 0

    lax.fori_loop(0, NCHUNK // NBUF, ring_body, 0)

    for b in range(NBUF):
        pltpu.make_async_copy(
            rows[b], out_hbm.at[pl.ds(base, CHUNK)], wsem[b]
        ).wait()


def kernel(input, table):
    idx = input.reshape(N)
    pe = _make_pe()
    out = _emb_lookup(idx, table, pe)
    return out.reshape(SEQ, BATCH, DIM)
